# baseline (speedup 1.0000x reference)
import functools

import jax
import jax.numpy as jnp
from jax.experimental import pallas as pl
from jax.experimental.pallas import tpu as pltpu

STATE = 128
N_ROWS = 100000
BLK = 10000


def _body(x_ref, stma_ref, stmb_ref, stmw_ref, ltma_ref, ltmb_ref, ltmw_ref,
          W1_ref, b1_ref, W2_ref, b2_ref, out_ref,
          accs_ref, accl_ref, m_ref, s_ref, w_scr, lw_scr):
    j = pl.program_id(0)
    nsteps = pl.num_programs(0)

    @pl.when(j == 0)
    def _init():
        accs_ref[...] = jnp.zeros_like(accs_ref)
        accl_ref[...] = jnp.zeros_like(accl_ref)
        m_ref[0] = jnp.float32(-1e30)
        s_ref[0] = jnp.float32(0.0)

    for jj in range(N_ROWS // BLK):
        @pl.when(j == jj)
        def _stage(jj=jj):
            w_scr[...] = stmw_ref[0:1, jj * BLK:(jj + 1) * BLK]
            lw_scr[...] = ltmw_ref[0:1, jj * BLK:(jj + 1) * BLK]

    x = x_ref[...]
    stm = jnp.concatenate([stma_ref[...], stmb_ref[...]], axis=0)
    w = w_scr[...]

    scores = jax.lax.dot_general(
        x, stm, (((1,), (1,)), ((), ())),
        preferred_element_type=jnp.float32)
    t = scores * w
    bm = jnp.max(t)
    m_old = m_ref[0]
    m_new = jnp.maximum(m_old, bm)
    c = jnp.exp(m_old - m_new)
    p = jnp.exp(t - m_new)
    s_ref[0] = s_ref[0] * c + jnp.sum(p)
    accs_ref[...] = accs_ref[...] * c + jax.lax.dot_general(
        p, stm, (((1,), (0,)), ((), ())),
        preferred_element_type=jnp.float32)
    m_ref[0] = m_new

    ltm = jnp.concatenate([ltma_ref[...], ltmb_ref[...]], axis=0)
    lw = lw_scr[...]
    accl_ref[...] += jax.lax.dot_general(
        lw, ltm, (((1,), (0,)), ((), ())),
        preferred_element_type=jnp.float32)

    @pl.when(j == nsteps - 1)
    def _fin():
        r_s = accs_ref[...] / s_ref[0]
        fused = jnp.concatenate([x, r_s, accl_ref[...]], axis=1)
        h = jnp.maximum(
            jnp.dot(fused, W1_ref[...], preferred_element_type=jnp.float32)
            + b1_ref[...], 0.0)
        out_ref[...] = (
            jnp.dot(h, W2_ref[...], preferred_element_type=jnp.float32)
            + b2_ref[...])


@jax.jit
def kernel(x_t, stm_emb, stm_weight, ltm_emb, ltm_weight, W1, b1, W2, b2):
    nsteps = N_ROWS // BLK
    out = pl.pallas_call(
        _body,
        grid=(nsteps,),
        in_specs=[
            pl.BlockSpec((1, STATE), lambda j: (0, 0)),
            pl.BlockSpec((BLK // 2, STATE), lambda j: (2 * j, 0)),
            pl.BlockSpec((BLK // 2, STATE), lambda j: (2 * j + 1, 0)),
            pl.BlockSpec((1, N_ROWS), lambda j: (0, 0)),
            pl.BlockSpec((BLK // 2, STATE), lambda j: (2 * j, 0)),
            pl.BlockSpec((BLK // 2, STATE), lambda j: (2 * j + 1, 0)),
            pl.BlockSpec((1, N_ROWS), lambda j: (0, 0)),
            pl.BlockSpec((3 * STATE, STATE), lambda j: (0, 0)),
            pl.BlockSpec((1, STATE), lambda j: (0, 0)),
            pl.BlockSpec((STATE, STATE), lambda j: (0, 0)),
            pl.BlockSpec((1, STATE), lambda j: (0, 0)),
        ],
        out_specs=pl.BlockSpec((1, STATE), lambda j: (0, 0)),
        out_shape=jax.ShapeDtypeStruct((1, STATE), jnp.float32),
        scratch_shapes=[
            pltpu.VMEM((1, STATE), jnp.float32),
            pltpu.VMEM((1, STATE), jnp.float32),
            pltpu.SMEM((1,), jnp.float32),
            pltpu.SMEM((1,), jnp.float32),
            pltpu.VMEM((1, BLK), jnp.float32),
            pltpu.VMEM((1, BLK), jnp.float32),
        ],
    )(
        x_t.reshape(1, STATE), stm_emb, stm_emb,
        stm_weight.reshape(1, N_ROWS),
        ltm_emb, ltm_emb, ltm_weight.reshape(1, N_ROWS),
        W1, b1.reshape(1, STATE), W2, b2.reshape(1, STATE),
    )
    return out.reshape(STATE)


# speedup vs baseline: 1.0430x; 1.0430x over previous
import jax
import jax.numpy as jnp
from jax.experimental import pallas as pl
from jax.experimental.pallas import tpu as pltpu

STATE = 128
N_ROWS = 100000
BLK = 10000


def _body(x_ref, stm_ref, stmw_ref, ltm_ref, ltmw_ref,
          W1_ref, b1_ref, W2_ref, b2_ref, out_ref,
          accs_ref, accl_ref, m_ref, s_ref, w_scr, lw_scr):
    j = pl.program_id(0)
    nsteps = pl.num_programs(0)

    @pl.when(j == 0)
    def _init():
        accs_ref[...] = jnp.zeros_like(accs_ref)
        accl_ref[...] = jnp.zeros_like(accl_ref)
        m_ref[0] = jnp.float32(-1e30)
        s_ref[0] = jnp.float32(0.0)

    for jj in range(N_ROWS // BLK):
        @pl.when(j == jj)
        def _stage(jj=jj):
            w_scr[...] = stmw_ref[0:1, jj * BLK:(jj + 1) * BLK]
            lw_scr[...] = ltmw_ref[0:1, jj * BLK:(jj + 1) * BLK]

    x = x_ref[...]
    stm = stm_ref[...]
    w = w_scr[...]

    scores = jax.lax.dot_general(
        x, stm, (((1,), (1,)), ((), ())),
        preferred_element_type=jnp.float32)
    t = scores * w
    bm = jnp.max(t)
    m_old = m_ref[0]
    m_new = jnp.maximum(m_old, bm)
    c = jnp.exp(m_old - m_new)
    p = jnp.exp(t - m_new)
    s_ref[0] = s_ref[0] * c + jnp.sum(p)
    accs_ref[...] = accs_ref[...] * c + jax.lax.dot_general(
        p, stm, (((1,), (0,)), ((), ())),
        preferred_element_type=jnp.float32)
    m_ref[0] = m_new

    ltm = ltm_ref[...]
    lw = lw_scr[...]
    accl_ref[...] += jax.lax.dot_general(
        lw, ltm, (((1,), (0,)), ((), ())),
        preferred_element_type=jnp.float32)

    @pl.when(j == nsteps - 1)
    def _fin():
        r_s = accs_ref[...] / s_ref[0]
        fused = jnp.concatenate([x, r_s, accl_ref[...]], axis=1)
        h = jnp.maximum(
            jnp.dot(fused, W1_ref[...], preferred_element_type=jnp.float32)
            + b1_ref[...], 0.0)
        out_ref[...] = (
            jnp.dot(h, W2_ref[...], preferred_element_type=jnp.float32)
            + b2_ref[...])


@jax.jit
def kernel(x_t, stm_emb, stm_weight, ltm_emb, ltm_weight, W1, b1, W2, b2):
    nsteps = N_ROWS // BLK
    out = pl.pallas_call(
        _body,
        grid=(nsteps,),
        in_specs=[
            pl.BlockSpec((1, STATE), lambda j: (0, 0)),
            pl.BlockSpec((BLK, STATE), lambda j: (j, 0)),
            pl.BlockSpec((1, N_ROWS), lambda j: (0, 0)),
            pl.BlockSpec((BLK, STATE), lambda j: (j, 0)),
            pl.BlockSpec((1, N_ROWS), lambda j: (0, 0)),
            pl.BlockSpec((3 * STATE, STATE), lambda j: (0, 0)),
            pl.BlockSpec((1, STATE), lambda j: (0, 0)),
            pl.BlockSpec((STATE, STATE), lambda j: (0, 0)),
            pl.BlockSpec((1, STATE), lambda j: (0, 0)),
        ],
        out_specs=pl.BlockSpec((1, STATE), lambda j: (0, 0)),
        out_shape=jax.ShapeDtypeStruct((1, STATE), jnp.float32),
        scratch_shapes=[
            pltpu.VMEM((1, STATE), jnp.float32),
            pltpu.VMEM((1, STATE), jnp.float32),
            pltpu.SMEM((1,), jnp.float32),
            pltpu.SMEM((1,), jnp.float32),
            pltpu.VMEM((1, BLK), jnp.float32),
            pltpu.VMEM((1, BLK), jnp.float32),
        ],
    )(
        x_t.reshape(1, STATE), stm_emb, stm_weight.reshape(1, N_ROWS),
        ltm_emb, ltm_weight.reshape(1, N_ROWS),
        W1, b1.reshape(1, STATE), W2, b2.reshape(1, STATE),
    )
    return out.reshape(STATE)
